# Initial kernel scaffold; baseline (speedup 1.0000x reference)
import jax
import jax.numpy as jnp
from jax.experimental import pallas as pl


def kernel(x, edge_index, Wl1, bl1, Wr1, Wl2, bl2, Wr2, Wl3, bl3, Wr3, Wl4, bl4, Wr4):
    raise NotImplementedError("write your pallas kernel here")


# speedup vs baseline: 21.5809x; 21.5809x over previous
import functools

import jax
import jax.numpy as jnp
from jax import lax
from jax.experimental import pallas as pl
from jax.experimental.pallas import tpu as pltpu
from jax.experimental.pallas import tpu_sc as plsc

N = 10000
NP = 10240
E = 320000
NC = 2
NS = 16
NW = NC * NS
EPW = E // NW
C = 2000
K = EPW // C
RPT = NP // NS


def _sc_agg(d, with_deg):
    mesh = plsc.VectorSubcoreMesh(core_axis_name="c", subcore_axis_name="s")
    out_type = [jax.ShapeDtypeStruct((NC, NP, d), jnp.float32)]
    scratch = [
        pltpu.VMEM((C,), jnp.int32),
        pltpu.VMEM((C,), jnp.int32),
        pltpu.VMEM((C, d), jnp.float32),
        pltpu.VMEM_SHARED((NP, d), jnp.float32),
        pltpu.SemaphoreType.DMA,
    ]
    if with_deg:
        out_type.append(jax.ShapeDtypeStruct((NC, NP, 16), jnp.float32))
        scratch += [
            pltpu.VMEM((C, 16), jnp.float32),
            pltpu.VMEM_SHARED((NP, 16), jnp.float32),
        ]

    def body(hp, srce, dste, zrows, zcol, onesc, *refs):
        if with_deg:
            agg, deg, src_v, dst_v, rows_v, acc, sem, ones_v, dacc = refs
        else:
            agg, src_v, dst_v, rows_v, acc, sem = refs
        c = lax.axis_index("c")
        s = lax.axis_index("s")
        pltpu.sync_copy(zrows.at[pl.ds(s * RPT, RPT)],
                        acc.at[pl.ds(s * RPT, RPT)])
        if with_deg:
            pltpu.sync_copy(onesc, ones_v)
            pltpu.sync_copy(zcol.at[pl.ds(s * RPT, RPT)],
                            dacc.at[pl.ds(s * RPT, RPT)])
        plsc.subcore_barrier()
        ebase = (c * NS + s) * EPW
        for k in range(K):
            b = ebase + k * C
            pltpu.sync_copy(srce.at[pl.ds(b, C)], src_v)
            pltpu.sync_copy(dste.at[pl.ds(b, C)], dst_v)
            pltpu.async_copy(hp.at[src_v], rows_v, sem).wait()
            pltpu.sync_copy(rows_v, acc.at[dst_v], add=True)
            if with_deg:
                pltpu.sync_copy(ones_v, dacc.at[dst_v], add=True)
        plsc.subcore_barrier()
        pltpu.sync_copy(acc.at[pl.ds(s * RPT, RPT)],
                        agg.at[c, pl.ds(s * RPT, RPT)])
        if with_deg:
            pltpu.sync_copy(dacc.at[pl.ds(s * RPT, RPT)],
                            deg.at[c, pl.ds(s * RPT, RPT)])

    return pl.kernel(body, mesh=mesh, out_type=out_type,
                     scratch_types=scratch,
                     compiler_params=pltpu.CompilerParams(
                         use_tc_tiling_on_sc=False))


_sc_agg_deg32 = _sc_agg(32, True)
_sc_agg16 = _sc_agg(16, False)


def _tc(body, out_shape, *args):
    return pl.pallas_call(body, out_shape=out_shape)(*args)


def _tc1(x_ref, w_ref, hp_ref, xr_ref):
    y = jnp.dot(x_ref[...], w_ref[...], preferred_element_type=jnp.float32)
    hp_ref[...] = y[:, :32]
    xr_ref[...] = y[:, 32:]


def _tc2(agg_ref, degp_ref, xr1_ref, bl1_ref, w2l_ref, w2r_ref,
         hp2_ref, xr2_ref, dinv_ref):
    dinv = 1.0 / jnp.maximum(degp_ref[0, :, 0:1] + degp_ref[1, :, 0:1], 1.0)
    mean1 = (agg_ref[0] + agg_ref[1]) * dinv
    h1 = jnp.maximum(mean1 + bl1_ref[...] + xr1_ref[...], 0.0)
    hp2_ref[...] = jnp.dot(h1, w2l_ref[...], preferred_element_type=jnp.float32)
    xr2_ref[...] = jnp.dot(h1, w2r_ref[...], preferred_element_type=jnp.float32)
    dinv_ref[...] = dinv


def _tc3(agg_ref, dinv_ref, xr2_ref, bl2_ref, wr3_ref, h2_ref, xr3_ref):
    h2 = (agg_ref[0] + agg_ref[1]) * dinv_ref[...] + bl2_ref[...] + xr2_ref[...]
    h2_ref[...] = h2
    xr3_ref[...] = jnp.dot(h2, wr3_ref[...], preferred_element_type=jnp.float32)


def _tc4(agg_ref, dinv_ref, xr3_ref, wl3_ref, bl3_ref, wl4_ref, wr4_ref,
         hp4_ref, xr4_ref):
    mean3 = (agg_ref[0] + agg_ref[1]) * dinv_ref[...]
    pre3 = (jnp.dot(mean3, wl3_ref[...], preferred_element_type=jnp.float32)
            + bl3_ref[...] + xr3_ref[...])
    h3 = jnp.where(pre3 > 0.0, pre3, jnp.exp(jnp.minimum(pre3, 0.0)) - 1.0)
    hp4_ref[...] = jnp.dot(h3, wl4_ref[...], preferred_element_type=jnp.float32)
    xr4_ref[...] = jnp.dot(h3, wr4_ref[...], preferred_element_type=jnp.float32)


def _tc5(agg_ref, dinv_ref, xr4_ref, bl4_ref, out_ref):
    mean4 = (agg_ref[0, :N, 0:1] + agg_ref[1, :N, 0:1]) * dinv_ref[:N]
    out_ref[...] = mean4 + bl4_ref[...] + xr4_ref[:N]


def kernel(x, edge_index, Wl1, bl1, Wr1, Wl2, bl2, Wr2, Wl3, bl3, Wr3,
           Wl4, bl4, Wr4):
    f32 = jnp.float32
    src = edge_index[0].astype(jnp.int32)
    dst = edge_index[1].astype(jnp.int32)

    W1cat = jnp.concatenate([Wl1.T, Wr1.T], axis=1)
    W2l = jnp.zeros((32, 16), f32).at[:, :9].set(Wl2.T)
    W2r = jnp.zeros((32, 16), f32).at[:, :9].set(Wr2.T)
    bl2p = jnp.zeros((1, 16), f32).at[0, :9].set(bl2)
    Wr3p = jnp.zeros((16, 32), f32).at[:9].set(Wr3.T)
    Wl3p = jnp.zeros((16, 32), f32).at[:9].set(Wl3.T)
    Wl4c = jnp.zeros((32, 16), f32).at[:, :1].set(Wl4.T)
    Wr4T = Wr4.T
    bl1r = bl1.reshape(1, 32)
    bl3r = bl3.reshape(1, 32)
    bl4r = bl4.reshape(1, 1)

    z32 = jnp.zeros((NP, 32), f32)
    z16 = jnp.zeros((NP, 16), f32)
    zcol = jnp.zeros((NP, 16), f32)
    onesc = jnp.ones((C, 16), f32)

    xp = jnp.pad(x, ((0, NP - N), (0, 0)))

    hp1, xr1 = _tc(_tc1,
                   (jax.ShapeDtypeStruct((NP, 32), f32),
                    jax.ShapeDtypeStruct((NP, 32), f32)),
                   xp, W1cat)
    agg1, degp = _sc_agg_deg32(hp1, src, dst, z32, zcol, onesc)

    hp2, xr2, dinv = _tc(_tc2,
                         (jax.ShapeDtypeStruct((NP, 16), f32),
                          jax.ShapeDtypeStruct((NP, 16), f32),
                          jax.ShapeDtypeStruct((NP, 1), f32)),
                         agg1, degp, xr1, bl1r, W2l, W2r)
    agg2 = _sc_agg16(hp2, src, dst, z16, zcol, onesc)[0]

    h2, xr3 = _tc(_tc3,
                  (jax.ShapeDtypeStruct((NP, 16), f32),
                   jax.ShapeDtypeStruct((NP, 32), f32)),
                  agg2, dinv, xr2, bl2p, Wr3p)
    agg3 = _sc_agg16(h2, src, dst, z16, zcol, onesc)[0]

    hp4, xr4 = _tc(_tc4,
                   (jax.ShapeDtypeStruct((NP, 16), f32),
                    jax.ShapeDtypeStruct((NP, 1), f32)),
                   agg3, dinv, xr3, Wl3p, bl3r, Wl4c, Wr4T)
    agg4 = _sc_agg16(hp4, src, dst, z16, zcol, onesc)[0]

    out = _tc(_tc5, jax.ShapeDtypeStruct((N, 1), f32),
              agg4, dinv, xr4, bl4r)
    return out


# speedup vs baseline: 25.8771x; 1.1991x over previous
import jax
import jax.numpy as jnp
from jax import lax
from jax.experimental import pallas as pl
from jax.experimental.pallas import tpu as pltpu
from jax.experimental.pallas import tpu_sc as plsc

N = 10000
NP = 10240
E = 320000
NC = 2
NS = 16
NW = NC * NS
EPW = E // NW
RPT = NP // NS
CD = 5000
KD = EPW // CD


def _sc_agg(d, c_chunk):
    k_ch = EPW // c_chunk
    mesh = plsc.VectorSubcoreMesh(core_axis_name="c", subcore_axis_name="s")
    out_type = jax.ShapeDtypeStruct((NC, NP, d), jnp.float32)
    scratch = [
        pltpu.VMEM((k_ch, c_chunk), jnp.int32),
        pltpu.VMEM((k_ch, c_chunk), jnp.int32),
        pltpu.VMEM((c_chunk, d), jnp.float32),
        pltpu.VMEM((c_chunk, d), jnp.float32),
        pltpu.VMEM_SHARED((NP, d), jnp.float32),
        pltpu.SemaphoreType.DMA,
        pltpu.SemaphoreType.DMA,
        pltpu.SemaphoreType.DMA,
        pltpu.SemaphoreType.DMA,
        pltpu.SemaphoreType.DMA,
        pltpu.SemaphoreType.DMA,
    ]

    def body(hp, ei, zfeed, agg, srci, dsti, rows0, rows1, acc,
             isrc, idst, g0, g1, s0, s1):
        rows = (rows0, rows1)
        gsem = (g0, g1)
        ssem = (s0, s1)
        c = lax.axis_index("c")
        s = lax.axis_index("s")
        pltpu.sync_copy(zfeed, acc.at[pl.ds(s * RPT, RPT)])
        ebase = (c * NS + s) * EPW
        ih = []
        for k in range(k_ch):
            b = ebase + k * c_chunk
            ih.append(pltpu.async_copy(ei.at[0, pl.ds(b, c_chunk)],
                                       srci.at[k], isrc))
            ih.append(pltpu.async_copy(ei.at[1, pl.ds(b, c_chunk)],
                                       dsti.at[k], idst))
        for h in ih:
            h.wait()
        plsc.subcore_barrier()
        gh = [None] * k_ch
        sh = [None] * k_ch
        for k in range(k_ch):
            if k >= 2:
                sh[k - 2].wait()
            gh[k] = pltpu.async_copy(hp.at[srci.at[k]], rows[k % 2],
                                     gsem[k % 2])
            if k >= 1:
                gh[k - 1].wait()
                sh[k - 1] = pltpu.async_copy(rows[(k - 1) % 2],
                                             acc.at[dsti.at[k - 1]],
                                             ssem[(k - 1) % 2], add=True)
        gh[k_ch - 1].wait()
        sh[k_ch - 1] = pltpu.async_copy(rows[(k_ch - 1) % 2],
                                        acc.at[dsti.at[k_ch - 1]],
                                        ssem[(k_ch - 1) % 2], add=True)
        if k_ch >= 2:
            sh[k_ch - 2].wait()
        sh[k_ch - 1].wait()
        plsc.subcore_barrier()
        pltpu.sync_copy(acc.at[pl.ds(s * RPT, RPT)],
                        agg.at[c, pl.ds(s * RPT, RPT)])

    return pl.kernel(body, mesh=mesh, out_type=out_type,
                     scratch_types=scratch,
                     compiler_params=pltpu.CompilerParams(
                         use_tc_tiling_on_sc=False))


def _sc_deg():
    mesh = plsc.VectorSubcoreMesh(core_axis_name="c", subcore_axis_name="s")
    out_type = jax.ShapeDtypeStruct((NC, NP, 16), jnp.float32)
    scratch = [
        pltpu.VMEM((KD, CD), jnp.int32),
        pltpu.VMEM((CD, 16), jnp.float32),
        pltpu.VMEM_SHARED((NP, 16), jnp.float32),
        pltpu.SemaphoreType.DMA,
        pltpu.SemaphoreType.DMA,
        pltpu.SemaphoreType.DMA,
    ]

    def body(ei, onesc, zfeed, degp, dsti, ones_v, dacc, isem, s0, s1):
        ssem = (s0, s1)
        c = lax.axis_index("c")
        s = lax.axis_index("s")
        pltpu.sync_copy(zfeed, dacc.at[pl.ds(s * RPT, RPT)])
        pltpu.sync_copy(onesc, ones_v)
        ebase = (c * NS + s) * EPW
        ih = [pltpu.async_copy(ei.at[1, pl.ds(ebase + k * CD, CD)],
                               dsti.at[k], isem) for k in range(KD)]
        for h in ih:
            h.wait()
        plsc.subcore_barrier()
        sh = [pltpu.async_copy(ones_v, dacc.at[dsti.at[k]], ssem[k % 2],
                               add=True) for k in range(KD)]
        for h in sh:
            h.wait()
        plsc.subcore_barrier()
        pltpu.sync_copy(dacc.at[pl.ds(s * RPT, RPT)],
                        degp.at[c, pl.ds(s * RPT, RPT)])

    return pl.kernel(body, mesh=mesh, out_type=out_type,
                     scratch_types=scratch,
                     compiler_params=pltpu.CompilerParams(
                         use_tc_tiling_on_sc=False))


_sc_agg32 = _sc_agg(32, 1000)
_sc_agg16 = _sc_agg(16, 2000)
_sc_deg_k = _sc_deg()


def _tc(body, out_shape, *args):
    return pl.pallas_call(body, out_shape=out_shape)(*args)


def _tc1(x_ref, w_ref, hp_ref, xr_ref):
    y = jnp.dot(x_ref[...], w_ref[...], preferred_element_type=jnp.float32)
    hp_ref[...] = y[:, :32]
    xr_ref[...] = y[:, 32:]


def _tc2(agg_ref, degp_ref, xr1_ref, bl1_ref, w2l_ref, w2r_ref,
         hp2_ref, xr2_ref, dinv_ref):
    dinv = 1.0 / jnp.maximum(degp_ref[0, :, 0:1] + degp_ref[1, :, 0:1], 1.0)
    mean1 = (agg_ref[0] + agg_ref[1]) * dinv
    h1 = jnp.maximum(mean1 + bl1_ref[...] + xr1_ref[...], 0.0)
    hp2_ref[...] = jnp.dot(h1, w2l_ref[...], preferred_element_type=jnp.float32)
    xr2_ref[...] = jnp.dot(h1, w2r_ref[...], preferred_element_type=jnp.float32)
    dinv_ref[...] = dinv


def _tc3(agg_ref, dinv_ref, xr2_ref, bl2_ref, wr3_ref, h2_ref, xr3_ref):
    h2 = (agg_ref[0] + agg_ref[1]) * dinv_ref[...] + bl2_ref[...] + xr2_ref[...]
    h2_ref[...] = h2
    xr3_ref[...] = jnp.dot(h2, wr3_ref[...], preferred_element_type=jnp.float32)


def _tc4(agg_ref, dinv_ref, xr3_ref, wl3_ref, bl3_ref, wl4_ref, wr4_ref,
         hp4_ref, xr4_ref):
    mean3 = (agg_ref[0] + agg_ref[1]) * dinv_ref[...]
    pre3 = (jnp.dot(mean3, wl3_ref[...], preferred_element_type=jnp.float32)
            + bl3_ref[...] + xr3_ref[...])
    h3 = jnp.where(pre3 > 0.0, pre3, jnp.exp(jnp.minimum(pre3, 0.0)) - 1.0)
    hp4_ref[...] = jnp.dot(h3, wl4_ref[...], preferred_element_type=jnp.float32)
    xr4_ref[...] = jnp.dot(h3, wr4_ref[...], preferred_element_type=jnp.float32)


def _tc5(agg_ref, dinv_ref, xr4_ref, bl4_ref, out_ref):
    mean4 = (agg_ref[0, :N, 0:1] + agg_ref[1, :N, 0:1]) * dinv_ref[:N]
    out_ref[...] = mean4 + bl4_ref[...] + xr4_ref[:N]


def kernel(x, edge_index, Wl1, bl1, Wr1, Wl2, bl2, Wr2, Wl3, bl3, Wr3,
           Wl4, bl4, Wr4):
    f32 = jnp.float32
    ei = edge_index.astype(jnp.int32)

    W1cat = jnp.concatenate([Wl1.T, Wr1.T], axis=1)
    W2l = jnp.zeros((32, 16), f32).at[:, :9].set(Wl2.T)
    W2r = jnp.zeros((32, 16), f32).at[:, :9].set(Wr2.T)
    bl2p = jnp.zeros((1, 16), f32).at[0, :9].set(bl2)
    Wr3p = jnp.zeros((16, 32), f32).at[:9].set(Wr3.T)
    Wl3p = jnp.zeros((16, 32), f32).at[:9].set(Wl3.T)
    Wl4c = jnp.zeros((32, 16), f32).at[:, :1].set(Wl4.T)
    Wr4T = Wr4.T
    bl1r = bl1.reshape(1, 32)
    bl3r = bl3.reshape(1, 32)
    bl4r = bl4.reshape(1, 1)

    z32 = jnp.zeros((RPT, 32), f32)
    z16 = jnp.zeros((RPT, 16), f32)
    onesc = jnp.ones((CD, 16), f32)

    xp = jnp.pad(x, ((0, NP - N), (0, 0)))

    degp = _sc_deg_k(ei, onesc, z16)

    hp1, xr1 = _tc(_tc1,
                   (jax.ShapeDtypeStruct((NP, 32), f32),
                    jax.ShapeDtypeStruct((NP, 32), f32)),
                   xp, W1cat)
    agg1 = _sc_agg32(hp1, ei, z32)

    hp2, xr2, dinv = _tc(_tc2,
                         (jax.ShapeDtypeStruct((NP, 16), f32),
                          jax.ShapeDtypeStruct((NP, 16), f32),
                          jax.ShapeDtypeStruct((NP, 1), f32)),
                         agg1, degp, xr1, bl1r, W2l, W2r)
    agg2 = _sc_agg16(hp2, ei, z16)

    h2, xr3 = _tc(_tc3,
                  (jax.ShapeDtypeStruct((NP, 16), f32),
                   jax.ShapeDtypeStruct((NP, 32), f32)),
                  agg2, dinv, xr2, bl2p, Wr3p)
    agg3 = _sc_agg16(h2, ei, z16)

    hp4, xr4 = _tc(_tc4,
                   (jax.ShapeDtypeStruct((NP, 16), f32),
                    jax.ShapeDtypeStruct((NP, 1), f32)),
                   agg3, dinv, xr3, Wl3p, bl3r, Wl4c, Wr4T)
    agg4 = _sc_agg16(hp4, ei, z16)

    out = _tc(_tc5, jax.ShapeDtypeStruct((N, 1), f32),
              agg4, dinv, xr4, bl4r)
    return out


# speedup vs baseline: 26.1571x; 1.0108x over previous
import jax
import jax.numpy as jnp
from jax import lax
from jax.experimental import pallas as pl
from jax.experimental.pallas import tpu as pltpu
from jax.experimental.pallas import tpu_sc as plsc

N = 10000
NP = 10240
E = 320000
NC = 2
NS = 16
NW = NC * NS
EPW = E // NW
RPT = NP // NS
CD = 5000
KD = EPW // CD


def _sc_agg(d, c_chunk):
    k_ch = EPW // c_chunk
    mesh = plsc.VectorSubcoreMesh(core_axis_name="c", subcore_axis_name="s")
    out_type = jax.ShapeDtypeStruct((NC, NP, d), jnp.float32)
    scratch = [
        pltpu.VMEM((k_ch, c_chunk), jnp.int32),
        pltpu.VMEM((k_ch, c_chunk), jnp.int32),
        pltpu.VMEM((c_chunk, d), jnp.float32),
        pltpu.VMEM((c_chunk, d), jnp.float32),
        pltpu.VMEM_SHARED((NP, d), jnp.float32),
        pltpu.SemaphoreType.DMA,
        pltpu.SemaphoreType.DMA,
        pltpu.SemaphoreType.DMA,
        pltpu.SemaphoreType.DMA,
        pltpu.SemaphoreType.DMA,
        pltpu.SemaphoreType.DMA,
    ]

    def body(hp, ei, zfeed, agg, srci, dsti, rows0, rows1, acc,
             isrc, idst, g0, g1, s0, s1):
        rows = (rows0, rows1)
        gsem = (g0, g1)
        ssem = (s0, s1)
        c = lax.axis_index("c")
        s = lax.axis_index("s")
        pltpu.sync_copy(zfeed, acc.at[pl.ds(s * RPT, RPT)])
        ebase = (c * NS + s) * EPW
        ih = []
        for k in range(k_ch):
            b = ebase + k * c_chunk
            ih.append(pltpu.async_copy(ei.at[0, pl.ds(b, c_chunk)],
                                       srci.at[k], isrc))
            ih.append(pltpu.async_copy(ei.at[1, pl.ds(b, c_chunk)],
                                       dsti.at[k], idst))
        for h in ih:
            h.wait()
        plsc.subcore_barrier()
        gh = [None] * k_ch
        sh = [None] * k_ch
        for k in range(k_ch):
            if k >= 2:
                sh[k - 2].wait()
            gh[k] = pltpu.async_copy(hp.at[srci.at[k]], rows[k % 2],
                                     gsem[k % 2])
            if k >= 1:
                gh[k - 1].wait()
                sh[k - 1] = pltpu.async_copy(rows[(k - 1) % 2],
                                             acc.at[dsti.at[k - 1]],
                                             ssem[(k - 1) % 2], add=True)
        gh[k_ch - 1].wait()
        sh[k_ch - 1] = pltpu.async_copy(rows[(k_ch - 1) % 2],
                                        acc.at[dsti.at[k_ch - 1]],
                                        ssem[(k_ch - 1) % 2], add=True)
        if k_ch >= 2:
            sh[k_ch - 2].wait()
        sh[k_ch - 1].wait()
        plsc.subcore_barrier()
        pltpu.sync_copy(acc.at[pl.ds(s * RPT, RPT)],
                        agg.at[c, pl.ds(s * RPT, RPT)])

    return pl.kernel(body, mesh=mesh, out_type=out_type,
                     scratch_types=scratch,
                     compiler_params=pltpu.CompilerParams(
                         use_tc_tiling_on_sc=False))


def _sc_deg():
    mesh = plsc.VectorSubcoreMesh(core_axis_name="c", subcore_axis_name="s")
    out_type = jax.ShapeDtypeStruct((NC, NP, 16), jnp.float32)
    scratch = [
        pltpu.VMEM((KD, CD), jnp.int32),
        pltpu.VMEM((CD, 16), jnp.float32),
        pltpu.VMEM_SHARED((NP, 16), jnp.float32),
        pltpu.SemaphoreType.DMA,
        pltpu.SemaphoreType.DMA,
        pltpu.SemaphoreType.DMA,
    ]

    def body(ei, onesc, zfeed, degp, dsti, ones_v, dacc, isem, s0, s1):
        ssem = (s0, s1)
        c = lax.axis_index("c")
        s = lax.axis_index("s")
        pltpu.sync_copy(zfeed, dacc.at[pl.ds(s * RPT, RPT)])
        pltpu.sync_copy(onesc, ones_v)
        ebase = (c * NS + s) * EPW
        ih = [pltpu.async_copy(ei.at[1, pl.ds(ebase + k * CD, CD)],
                               dsti.at[k], isem) for k in range(KD)]
        for h in ih:
            h.wait()
        plsc.subcore_barrier()
        sh = [pltpu.async_copy(ones_v, dacc.at[dsti.at[k]], ssem[k % 2],
                               add=True) for k in range(KD)]
        for h in sh:
            h.wait()
        plsc.subcore_barrier()
        pltpu.sync_copy(dacc.at[pl.ds(s * RPT, RPT)],
                        degp.at[c, pl.ds(s * RPT, RPT)])

    return pl.kernel(body, mesh=mesh, out_type=out_type,
                     scratch_types=scratch,
                     compiler_params=pltpu.CompilerParams(
                         use_tc_tiling_on_sc=False))


_sc_agg32 = _sc_agg(32, 1000)
_sc_agg16 = _sc_agg(16, 1000)
_sc_deg_k = _sc_deg()


def _tc(body, out_shape, *args):
    return pl.pallas_call(body, out_shape=out_shape)(*args)


def _tc1(x_ref, w_ref, hp_ref, xr_ref):
    y = jnp.dot(x_ref[...], w_ref[...], preferred_element_type=jnp.float32)
    hp_ref[...] = y[:, :32]
    xr_ref[...] = y[:, 32:]


def _tc2(agg_ref, degp_ref, xr1_ref, bl1_ref, w2l_ref, w2r_ref,
         hp2_ref, xr2_ref, dinv_ref):
    dinv = 1.0 / jnp.maximum(degp_ref[0, :, 0:1] + degp_ref[1, :, 0:1], 1.0)
    mean1 = (agg_ref[0] + agg_ref[1]) * dinv
    h1 = jnp.maximum(mean1 + bl1_ref[...] + xr1_ref[...], 0.0)
    hp2_ref[...] = jnp.dot(h1, w2l_ref[...], preferred_element_type=jnp.float32)
    xr2_ref[...] = jnp.dot(h1, w2r_ref[...], preferred_element_type=jnp.float32)
    dinv_ref[...] = dinv


def _tc3(agg_ref, dinv_ref, xr2_ref, bl2_ref, wr3_ref, h2_ref, xr3_ref):
    h2 = (agg_ref[0] + agg_ref[1]) * dinv_ref[...] + bl2_ref[...] + xr2_ref[...]
    h2_ref[...] = h2
    xr3_ref[...] = jnp.dot(h2, wr3_ref[...], preferred_element_type=jnp.float32)


def _tc4(agg_ref, dinv_ref, xr3_ref, wl3_ref, bl3_ref, wl4_ref, wr4_ref,
         hp4_ref, xr4_ref):
    mean3 = (agg_ref[0] + agg_ref[1]) * dinv_ref[...]
    pre3 = (jnp.dot(mean3, wl3_ref[...], preferred_element_type=jnp.float32)
            + bl3_ref[...] + xr3_ref[...])
    h3 = jnp.where(pre3 > 0.0, pre3, jnp.exp(jnp.minimum(pre3, 0.0)) - 1.0)
    hp4_ref[...] = jnp.dot(h3, wl4_ref[...], preferred_element_type=jnp.float32)
    xr4_ref[...] = jnp.dot(h3, wr4_ref[...], preferred_element_type=jnp.float32)


def _tc5(agg_ref, dinv_ref, xr4_ref, bl4_ref, out_ref):
    mean4 = (agg_ref[0, :N, 0:1] + agg_ref[1, :N, 0:1]) * dinv_ref[:N]
    out_ref[...] = mean4 + bl4_ref[...] + xr4_ref[:N]


def kernel(x, edge_index, Wl1, bl1, Wr1, Wl2, bl2, Wr2, Wl3, bl3, Wr3,
           Wl4, bl4, Wr4):
    f32 = jnp.float32
    ei = edge_index.astype(jnp.int32)

    W1cat = jnp.concatenate([Wl1.T, Wr1.T], axis=1)
    W2l = jnp.zeros((32, 16), f32).at[:, :9].set(Wl2.T)
    W2r = jnp.zeros((32, 16), f32).at[:, :9].set(Wr2.T)
    bl2p = jnp.zeros((1, 16), f32).at[0, :9].set(bl2)
    Wr3p = jnp.zeros((16, 32), f32).at[:9].set(Wr3.T)
    Wl3p = jnp.zeros((16, 32), f32).at[:9].set(Wl3.T)
    Wl4c = jnp.zeros((32, 16), f32).at[:, :1].set(Wl4.T)
    Wr4T = Wr4.T
    bl1r = bl1.reshape(1, 32)
    bl3r = bl3.reshape(1, 32)
    bl4r = bl4.reshape(1, 1)

    z32 = jnp.zeros((RPT, 32), f32)
    z16 = jnp.zeros((RPT, 16), f32)
    onesc = jnp.ones((CD, 16), f32)

    xp = jnp.pad(x, ((0, NP - N), (0, 0)))

    degp = _sc_deg_k(ei, onesc, z16)

    hp1, xr1 = _tc(_tc1,
                   (jax.ShapeDtypeStruct((NP, 32), f32),
                    jax.ShapeDtypeStruct((NP, 32), f32)),
                   xp, W1cat)
    agg1 = _sc_agg32(hp1, ei, z32)

    hp2, xr2, dinv = _tc(_tc2,
                         (jax.ShapeDtypeStruct((NP, 16), f32),
                          jax.ShapeDtypeStruct((NP, 16), f32),
                          jax.ShapeDtypeStruct((NP, 1), f32)),
                         agg1, degp, xr1, bl1r, W2l, W2r)
    agg2 = _sc_agg16(hp2, ei, z16)

    h2, xr3 = _tc(_tc3,
                  (jax.ShapeDtypeStruct((NP, 16), f32),
                   jax.ShapeDtypeStruct((NP, 32), f32)),
                  agg2, dinv, xr2, bl2p, Wr3p)
    agg3 = _sc_agg16(h2, ei, z16)

    hp4, xr4 = _tc(_tc4,
                   (jax.ShapeDtypeStruct((NP, 16), f32),
                    jax.ShapeDtypeStruct((NP, 1), f32)),
                   agg3, dinv, xr3, Wl3p, bl3r, Wl4c, Wr4T)
    agg4 = _sc_agg16(hp4, ei, z16)

    out = _tc(_tc5, jax.ShapeDtypeStruct((N, 1), f32),
              agg4, dinv, xr4, bl4r)
    return out
